# Initial kernel scaffold; baseline (speedup 1.0000x reference)
import jax
import jax.numpy as jnp
from jax.experimental import pallas as pl


def kernel(x, edge_index, We1, be1, We2, be2, Wg1, bg1, Wg2, bg2, Wp1, bp1, Wp2, bp2):
    raise NotImplementedError("write your pallas kernel here")


# speedup vs baseline: 9.9233x; 9.9233x over previous
import functools

import jax
import jax.numpy as jnp
from jax import lax
from jax.experimental import pallas as pl
from jax.experimental.pallas import tpu as pltpu
from jax.experimental.pallas import tpu_sc as plsc

_N = 100000
_E = 1600000
_NPAD = 100096
_EROWS = 12544
_ROWS_PER_TILE = _EROWS // 16
_IB = 112
_WOUT = _N // 16
_ZROWS = _NPAD // 16

_RB = 2000


def _sc_scatter(g_q, src_rows, dst_rows, zeros_blk):
    mesh = plsc.VectorSubcoreMesh(core_axis_name="c", subcore_axis_name="s")

    @functools.partial(
        pl.kernel,
        out_type=[jax.ShapeDtypeStruct((_NPAD, 8), jnp.float32)] * 4,
        mesh=mesh,
        scratch_types=[
            pltpu.VMEM((_IB, 128), jnp.int32),
            pltpu.VMEM((_IB, 128), jnp.int32),
            pltpu.VMEM((128, 8), jnp.float32),
            pltpu.VMEM_SHARED((_NPAD, 8), jnp.float32),
            pltpu.SemaphoreType.DMA,
        ],
        compiler_params=pltpu.CompilerParams(use_tc_tiling_on_sc=False),
    )
    def k(g0_hbm, g1_hbm, g2_hbm, g3_hbm, zeros_hbm, srcr_hbm, dstr_hbm,
          out0, out1, out2, out3, src_v, dst_v, rows_v, acc, sem):
        c = lax.axis_index("c")
        s = lax.axis_index("s")

        def run(table, out_ref):
            pltpu.sync_copy(zeros_hbm, acc.at[pl.ds(s * _ZROWS, _ZROWS)])
            plsc.subcore_barrier()
            row0 = s * _ROWS_PER_TILE

            def stage(si, carry):
                sb = row0 + si * _IB
                pltpu.sync_copy(srcr_hbm.at[pl.ds(sb, _IB)], src_v)
                pltpu.sync_copy(dstr_hbm.at[pl.ds(sb, _IB)], dst_v)

                def one(j, c2):
                    pltpu.async_copy(table.at[src_v.at[j]], rows_v, sem).wait()
                    pltpu.sync_copy(rows_v, acc.at[dst_v.at[j]], add=True)
                    return c2
                lax.fori_loop(0, _IB, one, 0)
                return carry
            lax.fori_loop(0, _ROWS_PER_TILE // _IB, stage, 0)
            plsc.subcore_barrier()
            pltpu.sync_copy(acc.at[pl.ds(s * _ZROWS, _ZROWS)],
                            out_ref.at[pl.ds(s * _ZROWS, _ZROWS)])

        tables = (g0_hbm, g1_hbm, g2_hbm, g3_hbm)
        outs = (out0, out1, out2, out3)
        for r in range(2):
            @pl.when(c == 0)
            def _(r=r):
                run(tables[r], outs[r])

            @pl.when(c == 1)
            def _(r=r):
                run(tables[2 + r], outs[2 + r])

    return k(g_q[0], g_q[1], g_q[2], g_q[3], zeros_blk, src_rows, dst_rows)


def _sc_deg(dst_rows):
    mesh = plsc.VectorSubcoreMesh(core_axis_name="c", subcore_axis_name="s")
    rows_per_core = _EROWS // 2
    rows_per_tile = rows_per_core // 16
    ib = 56

    @functools.partial(
        pl.kernel,
        out_type=[jax.ShapeDtypeStruct((_NPAD,), jnp.float32)] * 2,
        mesh=mesh,
        scratch_types=[
            pltpu.VMEM((ib, 128), jnp.int32),
            pltpu.VMEM((128,), jnp.float32),
            pltpu.VMEM((128,), jnp.float32),
            pltpu.VMEM((_ZROWS,), jnp.float32),
            pltpu.VMEM_SHARED((_NPAD,), jnp.float32),
            pltpu.SemaphoreType.DMA,
        ],
        compiler_params=pltpu.CompilerParams(use_tc_tiling_on_sc=False),
    )
    def k(dstr_hbm, out0, out1, dst_v, ones_v, zbuf, wbuf, acc, sem):
        c = lax.axis_index("c")
        s = lax.axis_index("s")

        for i in range(8):
            ones_v[pl.ds(16 * i, 16)] = jnp.ones((16,), jnp.float32)
            zbuf[pl.ds(16 * i, 16)] = jnp.zeros((16,), jnp.float32)

        zbase = s * _ZROWS
        def zchunk(t, carry):
            pltpu.sync_copy(zbuf, acc.at[pl.ds(zbase + t * 128, 128)])
            return carry
        lax.fori_loop(0, 48, zchunk, 0)
        pltpu.sync_copy(zbuf.at[pl.ds(0, 112)],
                        acc.at[pl.ds(zbase + 48 * 128, 112)])
        plsc.subcore_barrier()

        row0 = c * rows_per_core + s * rows_per_tile

        def stage(si, carry):
            sb = row0 + si * ib
            pltpu.sync_copy(dstr_hbm.at[pl.ds(sb, ib)], dst_v)

            def one(j, c2):
                pltpu.sync_copy(ones_v, acc.at[dst_v.at[j]], add=True)
                return c2
            lax.fori_loop(0, ib, one, 0)
            return carry
        lax.fori_loop(0, rows_per_tile // ib, stage, 0)
        plsc.subcore_barrier()

        pltpu.sync_copy(acc.at[pl.ds(s * _ZROWS, _ZROWS)], wbuf)

        @pl.when(c == 0)
        def _():
            pltpu.sync_copy(wbuf, out0.at[pl.ds(s * _ZROWS, _ZROWS)])

        @pl.when(c == 1)
        def _():
            pltpu.sync_copy(wbuf, out1.at[pl.ds(s * _ZROWS, _ZROWS)])

    return k(dst_rows)


def _full(shape):
    return pl.BlockSpec(shape, lambda i: (0, 0))


def _rows(width):
    return pl.BlockSpec((_RB, width), lambda i: (i, 0))


def _split4(g):
    return [g[:, 8 * q:8 * (q + 1)] for q in range(4)]


def _tc_embed(x, We1, be1, We2, be2, Wg1, deg0, deg1):
    def body(x_r, w1_r, b1_r, w2_r, b2_r, wg_r, d0_r, d1_r,
             o0_r, o1_r, o2_r, o3_r):
        h = jnp.tanh(jnp.dot(x_r[...], w1_r[...],
                             preferred_element_type=jnp.float32) + b1_r[...])
        h = jnp.tanh(jnp.dot(h, w2_r[...],
                             preferred_element_type=jnp.float32) + b2_r[...])
        p = jnp.dot(h, wg_r[...], preferred_element_type=jnp.float32)
        dinv = lax.rsqrt(d0_r[...] + d1_r[...] + 1.0)
        g = p * dinv
        for q, o_r in enumerate((o0_r, o1_r, o2_r, o3_r)):
            o_r[...] = g[:, 8 * q:8 * (q + 1)]

    return pl.pallas_call(
        body,
        grid=(_N // _RB,),
        in_specs=[
            _rows(6), _full((6, 64)), _full((1, 64)), _full((64, 32)),
            _full((1, 32)), _full((32, 32)), _rows(1), _rows(1),
        ],
        out_specs=[_rows(8)] * 4,
        out_shape=[jax.ShapeDtypeStruct((_N, 8), jnp.float32)] * 4,
    )(x, We1, be1, We2, be2, Wg1, deg0, deg1)


def _tc_mid(S_q, g_q, deg0, deg1, bg, Wg2):
    def body(s0_r, s1_r, s2_r, s3_r, g0_r, g1_r, g2_r, g3_r,
             d0_r, d1_r, bg_r, wg_r, o0_r, o1_r, o2_r, o3_r):
        dinv = lax.rsqrt(d0_r[...] + d1_r[...] + 1.0)
        s_rs = (s0_r, s1_r, s2_r, s3_r)
        g_rs = (g0_r, g1_r, g2_r, g3_r)
        p = None
        for q in range(4):
            hq = jnp.maximum((s_rs[q][...] + g_rs[q][...]) * dinv
                             + bg_r[...][:, 8 * q:8 * (q + 1)], 0.0)
            t = jnp.dot(hq, wg_r[...][8 * q:8 * (q + 1), :],
                        preferred_element_type=jnp.float32)
            p = t if p is None else p + t
        g = p * dinv
        for q, o_r in enumerate((o0_r, o1_r, o2_r, o3_r)):
            o_r[...] = g[:, 8 * q:8 * (q + 1)]

    return pl.pallas_call(
        body,
        grid=(_N // _RB,),
        in_specs=[_rows(8)] * 8 + [
            _rows(1), _rows(1), _full((1, 32)), _full((32, 32)),
        ],
        out_specs=[_rows(8)] * 4,
        out_shape=[jax.ShapeDtypeStruct((_N, 8), jnp.float32)] * 4,
    )(*S_q, *g_q, deg0, deg1, bg, Wg2)


def _tc_head(S_q, g_q, deg0, deg1, bg, Wp1, bp1, Wp2, bp2):
    def body(s0_r, s1_r, s2_r, s3_r, g0_r, g1_r, g2_r, g3_r,
             d0_r, d1_r, bg_r, w1_r, b1_r, w2_r, b2_r, out_r):
        dinv = lax.rsqrt(d0_r[...] + d1_r[...] + 1.0)
        s_rs = (s0_r, s1_r, s2_r, s3_r)
        g_rs = (g0_r, g1_r, g2_r, g3_r)
        z = None
        for q in range(4):
            hq = jnp.maximum((s_rs[q][...] + g_rs[q][...]) * dinv
                             + bg_r[...][:, 8 * q:8 * (q + 1)], 0.0)
            t = jnp.dot(hq, w1_r[...][8 * q:8 * (q + 1), :],
                        preferred_element_type=jnp.float32)
            z = t if z is None else z + t
        z = jnp.tanh(z + b1_r[...])
        out_r[...] = jnp.tanh(jnp.dot(z, w2_r[...],
                                      preferred_element_type=jnp.float32)
                              + b2_r[...])

    return pl.pallas_call(
        body,
        grid=(_N // _RB,),
        in_specs=[_rows(8)] * 8 + [
            _rows(1), _rows(1), _full((1, 32)), _full((32, 32)),
            _full((1, 32)), _full((32, 1)), _full((1, 1)),
        ],
        out_specs=[_rows(1)],
        out_shape=[jax.ShapeDtypeStruct((_N, 1), jnp.float32)],
    )(*S_q, *g_q, deg0, deg1, bg, Wp1, bp1, Wp2, bp2)


def kernel(x, edge_index, We1, be1, We2, be2, Wg1, bg1, Wg2, bg2,
           Wp1, bp1, Wp2, bp2):
    src = edge_index[0]
    dst = edge_index[1]

    npad = _EROWS * 128 - _E
    fill = jnp.arange(npad, dtype=jnp.int32) % 64
    src_rows = jnp.concatenate([src, fill]).reshape(_EROWS, 128)
    dst_rows = jnp.concatenate([dst, _N + fill]).reshape(_EROWS, 128)

    deg0p, deg1p = _sc_deg(dst_rows)
    deg0 = deg0p[:_N].reshape(_N, 1)
    deg1 = deg1p[:_N].reshape(_N, 1)

    be1r = be1.reshape(1, 64)
    be2r = be2.reshape(1, 32)
    bg1r = bg1.reshape(1, 32)
    bg2r = bg2.reshape(1, 32)
    bp1r = bp1.reshape(1, 32)
    bp2r = bp2.reshape(1, 1)

    zeros_blk = jnp.zeros((_ZROWS, 8), jnp.float32)

    g1_q = _tc_embed(x, We1, be1r, We2, be2r, Wg1, deg0, deg1)
    S1_q = _sc_scatter(g1_q, src_rows, dst_rows, zeros_blk)
    S1_q = [Sq[:_N] for Sq in S1_q]
    g2_q = _tc_mid(S1_q, g1_q, deg0, deg1, bg1r, Wg2)
    S2_q = _sc_scatter(g2_q, src_rows, dst_rows, zeros_blk)
    S2_q = [Sq[:_N] for Sq in S2_q]
    out, = _tc_head(S2_q, g2_q, deg0, deg1, bg2r, Wp1, bp1r, Wp2, bp2r)
    return out


# speedup vs baseline: 19.7117x; 1.9864x over previous
import functools

import jax
import jax.numpy as jnp
from jax import lax
from jax.experimental import pallas as pl
from jax.experimental.pallas import tpu as pltpu
from jax.experimental.pallas import tpu_sc as plsc

_N = 100000
_E = 1600000
_NPAD = 100096
_EROWS = 12544
_ROWS_PER_TILE = _EROWS // 16
_IB = 112
_WOUT = _N // 16
_ZROWS = _NPAD // 16

_RB = 2000


def _sc_scatter(g_q, src_rows, dst_rows, zeros_blk):
    mesh = plsc.VectorSubcoreMesh(core_axis_name="c", subcore_axis_name="s")

    @functools.partial(
        pl.kernel,
        out_type=[jax.ShapeDtypeStruct((_NPAD, 8), jnp.float32)] * 4,
        mesh=mesh,
        scratch_types=[
            pltpu.VMEM((_IB, 128), jnp.int32),
            pltpu.VMEM((_IB, 128), jnp.int32),
            pltpu.VMEM((16, 128, 8), jnp.float32),
            pltpu.VMEM_SHARED((_NPAD, 8), jnp.float32),
            pltpu.SemaphoreType.DMA,
            pltpu.SemaphoreType.DMA,
            pltpu.SemaphoreType.DMA,
            pltpu.SemaphoreType.DMA,
        ],
        compiler_params=pltpu.CompilerParams(use_tc_tiling_on_sc=False),
    )
    def k(g0_hbm, g1_hbm, g2_hbm, g3_hbm, zeros_hbm, srcr_hbm, dstr_hbm,
          out0, out1, out2, out3, src_v, dst_v, rows_v, acc,
          sga, sgb, ssa, ssb):
        c = lax.axis_index("c")
        s = lax.axis_index("s")

        def run(table, out_ref):
            pltpu.sync_copy(zeros_hbm, acc.at[pl.ds(s * _ZROWS, _ZROWS)])
            plsc.subcore_barrier()
            row0 = s * _ROWS_PER_TILE

            def ig(g, base, sem):
                for b in range(8):
                    pltpu.async_copy(table.at[src_v.at[g * 8 + b]],
                                     rows_v.at[base + b], sem)

            def dg(g, base, sem):
                for b in range(8):
                    pltpu.make_async_copy(table.at[src_v.at[g * 8 + b]],
                                          rows_v.at[base + b], sem).wait()

            def isc(g, base, sem):
                for b in range(8):
                    pltpu.async_copy(rows_v.at[base + b],
                                     acc.at[dst_v.at[g * 8 + b]], sem,
                                     add=True)

            def dsc(g, base, sem):
                for b in range(8):
                    pltpu.make_async_copy(rows_v.at[base + b],
                                          acc.at[dst_v.at[g * 8 + b]],
                                          sem).wait()

            def phase(g, cur_base, sg_cur, ss_cur, oth_base, ss_oth):
                dsc(g - 1, oth_base, ss_oth)
                ig(g + 1, oth_base, sgb if cur_base == 0 else sga)
                dg(g, cur_base, sg_cur)
                isc(g, cur_base, ss_cur)

            def stage(si, carry):
                sb = row0 + si * _IB
                pltpu.sync_copy(srcr_hbm.at[pl.ds(sb, _IB)], src_v)
                pltpu.sync_copy(dstr_hbm.at[pl.ds(sb, _IB)], dst_v)

                ig(0, 0, sga)
                ig(1, 8, sgb)
                dg(0, 0, sga)
                isc(0, 0, ssa)

                def pair(t, c2):
                    phase(2 * t + 1, 8, sgb, ssb, 0, ssa)
                    phase(2 * t + 2, 0, sga, ssa, 8, ssb)
                    return c2
                lax.fori_loop(0, (_IB // 16) - 1, pair, 0)

                gl = _IB // 8 - 1
                dsc(gl - 1, 0, ssa)
                dg(gl, 8, sgb)
                isc(gl, 8, ssb)
                dsc(gl, 8, ssb)
                return carry
            lax.fori_loop(0, _ROWS_PER_TILE // _IB, stage, 0)
            plsc.subcore_barrier()
            pltpu.sync_copy(acc.at[pl.ds(s * _ZROWS, _ZROWS)],
                            out_ref.at[pl.ds(s * _ZROWS, _ZROWS)])

        tables = (g0_hbm, g1_hbm, g2_hbm, g3_hbm)
        outs = (out0, out1, out2, out3)
        for r in range(2):
            @pl.when(c == 0)
            def _(r=r):
                run(tables[r], outs[r])

            @pl.when(c == 1)
            def _(r=r):
                run(tables[2 + r], outs[2 + r])

    return k(g_q[0], g_q[1], g_q[2], g_q[3], zeros_blk, src_rows, dst_rows)


def _sc_deg(dst_rows):
    mesh = plsc.VectorSubcoreMesh(core_axis_name="c", subcore_axis_name="s")
    rows_per_core = _EROWS // 2
    rows_per_tile = rows_per_core // 16
    ib = 56

    @functools.partial(
        pl.kernel,
        out_type=[jax.ShapeDtypeStruct((_NPAD,), jnp.float32)] * 2,
        mesh=mesh,
        scratch_types=[
            pltpu.VMEM((ib, 128), jnp.int32),
            pltpu.VMEM((128,), jnp.float32),
            pltpu.VMEM((128,), jnp.float32),
            pltpu.VMEM((_ZROWS,), jnp.float32),
            pltpu.VMEM_SHARED((_NPAD,), jnp.float32),
            pltpu.SemaphoreType.DMA,
        ],
        compiler_params=pltpu.CompilerParams(use_tc_tiling_on_sc=False),
    )
    def k(dstr_hbm, out0, out1, dst_v, ones_v, zbuf, wbuf, acc, sem):
        c = lax.axis_index("c")
        s = lax.axis_index("s")

        for i in range(8):
            ones_v[pl.ds(16 * i, 16)] = jnp.ones((16,), jnp.float32)
            zbuf[pl.ds(16 * i, 16)] = jnp.zeros((16,), jnp.float32)

        zbase = s * _ZROWS
        def zchunk(t, carry):
            pltpu.sync_copy(zbuf, acc.at[pl.ds(zbase + t * 128, 128)])
            return carry
        lax.fori_loop(0, 48, zchunk, 0)
        pltpu.sync_copy(zbuf.at[pl.ds(0, 112)],
                        acc.at[pl.ds(zbase + 48 * 128, 112)])
        plsc.subcore_barrier()

        row0 = c * rows_per_core + s * rows_per_tile

        def stage(si, carry):
            sb = row0 + si * ib
            pltpu.sync_copy(dstr_hbm.at[pl.ds(sb, ib)], dst_v)

            def one(j, c2):
                pltpu.sync_copy(ones_v, acc.at[dst_v.at[j]], add=True)
                return c2
            lax.fori_loop(0, ib, one, 0)
            return carry
        lax.fori_loop(0, rows_per_tile // ib, stage, 0)
        plsc.subcore_barrier()

        pltpu.sync_copy(acc.at[pl.ds(s * _ZROWS, _ZROWS)], wbuf)

        @pl.when(c == 0)
        def _():
            pltpu.sync_copy(wbuf, out0.at[pl.ds(s * _ZROWS, _ZROWS)])

        @pl.when(c == 1)
        def _():
            pltpu.sync_copy(wbuf, out1.at[pl.ds(s * _ZROWS, _ZROWS)])

    return k(dst_rows)


def _full(shape):
    return pl.BlockSpec(shape, lambda i: (0, 0))


def _rows(width):
    return pl.BlockSpec((_RB, width), lambda i: (i, 0))


def _split4(g):
    return [g[:, 8 * q:8 * (q + 1)] for q in range(4)]


def _tc_embed(x, We1, be1, We2, be2, Wg1, deg0, deg1):
    def body(x_r, w1_r, b1_r, w2_r, b2_r, wg_r, d0_r, d1_r,
             o0_r, o1_r, o2_r, o3_r):
        h = jnp.tanh(jnp.dot(x_r[...], w1_r[...],
                             preferred_element_type=jnp.float32) + b1_r[...])
        h = jnp.tanh(jnp.dot(h, w2_r[...],
                             preferred_element_type=jnp.float32) + b2_r[...])
        p = jnp.dot(h, wg_r[...], preferred_element_type=jnp.float32)
        dinv = lax.rsqrt(d0_r[...] + d1_r[...] + 1.0)
        g = p * dinv
        for q, o_r in enumerate((o0_r, o1_r, o2_r, o3_r)):
            o_r[...] = g[:, 8 * q:8 * (q + 1)]

    return pl.pallas_call(
        body,
        grid=(_N // _RB,),
        in_specs=[
            _rows(6), _full((6, 64)), _full((1, 64)), _full((64, 32)),
            _full((1, 32)), _full((32, 32)), _rows(1), _rows(1),
        ],
        out_specs=[_rows(8)] * 4,
        out_shape=[jax.ShapeDtypeStruct((_N, 8), jnp.float32)] * 4,
    )(x, We1, be1, We2, be2, Wg1, deg0, deg1)


def _tc_mid(S_q, g_q, deg0, deg1, bg, Wg2):
    def body(s0_r, s1_r, s2_r, s3_r, g0_r, g1_r, g2_r, g3_r,
             d0_r, d1_r, bg_r, wg_r, o0_r, o1_r, o2_r, o3_r):
        dinv = lax.rsqrt(d0_r[...] + d1_r[...] + 1.0)
        s_rs = (s0_r, s1_r, s2_r, s3_r)
        g_rs = (g0_r, g1_r, g2_r, g3_r)
        p = None
        for q in range(4):
            hq = jnp.maximum((s_rs[q][...] + g_rs[q][...]) * dinv
                             + bg_r[...][:, 8 * q:8 * (q + 1)], 0.0)
            t = jnp.dot(hq, wg_r[...][8 * q:8 * (q + 1), :],
                        preferred_element_type=jnp.float32)
            p = t if p is None else p + t
        g = p * dinv
        for q, o_r in enumerate((o0_r, o1_r, o2_r, o3_r)):
            o_r[...] = g[:, 8 * q:8 * (q + 1)]

    return pl.pallas_call(
        body,
        grid=(_N // _RB,),
        in_specs=[_rows(8)] * 8 + [
            _rows(1), _rows(1), _full((1, 32)), _full((32, 32)),
        ],
        out_specs=[_rows(8)] * 4,
        out_shape=[jax.ShapeDtypeStruct((_N, 8), jnp.float32)] * 4,
    )(*S_q, *g_q, deg0, deg1, bg, Wg2)


def _tc_head(S_q, g_q, deg0, deg1, bg, Wp1, bp1, Wp2, bp2):
    def body(s0_r, s1_r, s2_r, s3_r, g0_r, g1_r, g2_r, g3_r,
             d0_r, d1_r, bg_r, w1_r, b1_r, w2_r, b2_r, out_r):
        dinv = lax.rsqrt(d0_r[...] + d1_r[...] + 1.0)
        s_rs = (s0_r, s1_r, s2_r, s3_r)
        g_rs = (g0_r, g1_r, g2_r, g3_r)
        z = None
        for q in range(4):
            hq = jnp.maximum((s_rs[q][...] + g_rs[q][...]) * dinv
                             + bg_r[...][:, 8 * q:8 * (q + 1)], 0.0)
            t = jnp.dot(hq, w1_r[...][8 * q:8 * (q + 1), :],
                        preferred_element_type=jnp.float32)
            z = t if z is None else z + t
        z = jnp.tanh(z + b1_r[...])
        out_r[...] = jnp.tanh(jnp.dot(z, w2_r[...],
                                      preferred_element_type=jnp.float32)
                              + b2_r[...])

    return pl.pallas_call(
        body,
        grid=(_N // _RB,),
        in_specs=[_rows(8)] * 8 + [
            _rows(1), _rows(1), _full((1, 32)), _full((32, 32)),
            _full((1, 32)), _full((32, 1)), _full((1, 1)),
        ],
        out_specs=[_rows(1)],
        out_shape=[jax.ShapeDtypeStruct((_N, 1), jnp.float32)],
    )(*S_q, *g_q, deg0, deg1, bg, Wp1, bp1, Wp2, bp2)


def kernel(x, edge_index, We1, be1, We2, be2, Wg1, bg1, Wg2, bg2,
           Wp1, bp1, Wp2, bp2):
    src = edge_index[0]
    dst = edge_index[1]

    npad = _EROWS * 128 - _E
    fill = jnp.arange(npad, dtype=jnp.int32) % 64
    src_rows = jnp.concatenate([src, fill]).reshape(_EROWS, 128)
    dst_rows = jnp.concatenate([dst, _N + fill]).reshape(_EROWS, 128)

    deg0p, deg1p = _sc_deg(dst_rows)
    deg0 = deg0p[:_N].reshape(_N, 1)
    deg1 = deg1p[:_N].reshape(_N, 1)

    be1r = be1.reshape(1, 64)
    be2r = be2.reshape(1, 32)
    bg1r = bg1.reshape(1, 32)
    bg2r = bg2.reshape(1, 32)
    bp1r = bp1.reshape(1, 32)
    bp2r = bp2.reshape(1, 1)

    zeros_blk = jnp.zeros((_ZROWS, 8), jnp.float32)

    g1_q = _tc_embed(x, We1, be1r, We2, be2r, Wg1, deg0, deg1)
    S1_q = _sc_scatter(g1_q, src_rows, dst_rows, zeros_blk)
    S1_q = [Sq[:_N] for Sq in S1_q]
    g2_q = _tc_mid(S1_q, g1_q, deg0, deg1, bg1r, Wg2)
    S2_q = _sc_scatter(g2_q, src_rows, dst_rows, zeros_blk)
    S2_q = [Sq[:_N] for Sq in S2_q]
    out, = _tc_head(S2_q, g2_q, deg0, deg1, bg2r, Wp1, bp1r, Wp2, bp2r)
    return out


# speedup vs baseline: 22.8988x; 1.1617x over previous
import functools

import jax
import jax.numpy as jnp
from jax import lax
from jax.experimental import pallas as pl
from jax.experimental.pallas import tpu as pltpu
from jax.experimental.pallas import tpu_sc as plsc

_N = 100000
_E = 1600000
_NPAD = 100096
_EROWS = 12544
_ROWS_PER_TILE = _EROWS // 16
_IB = 112
_WOUT = _N // 16
_ZROWS = _NPAD // 16

_RB = 2000


def _sc_scatter(g_q, src_rows, dst_rows, zeros_blk):
    mesh = plsc.VectorSubcoreMesh(core_axis_name="c", subcore_axis_name="s")

    @functools.partial(
        pl.kernel,
        out_type=jax.ShapeDtypeStruct((4, _NPAD, 8), jnp.float32),
        mesh=mesh,
        scratch_types=[
            pltpu.VMEM((_IB, 128), jnp.int32),
            pltpu.VMEM((_IB, 128), jnp.int32),
            pltpu.VMEM((16, 128, 8), jnp.float32),
            pltpu.VMEM_SHARED((_NPAD, 8), jnp.float32),
            pltpu.SemaphoreType.DMA,
            pltpu.SemaphoreType.DMA,
            pltpu.SemaphoreType.DMA,
            pltpu.SemaphoreType.DMA,
        ],
        compiler_params=pltpu.CompilerParams(use_tc_tiling_on_sc=False),
    )
    def k(g_hbm, zeros_hbm, srcr_hbm, dstr_hbm, out_hbm,
          src_v, dst_v, rows_v, acc, sga, sgb, ssa, ssb):
        c = lax.axis_index("c")
        s = lax.axis_index("s")

        def run(table, out_ref):
            pltpu.sync_copy(zeros_hbm, acc.at[pl.ds(s * _ZROWS, _ZROWS)])
            plsc.subcore_barrier()
            row0 = s * _ROWS_PER_TILE

            def ig(g, base, sem):
                for b in range(8):
                    pltpu.async_copy(table.at[src_v.at[g * 8 + b]],
                                     rows_v.at[base + b], sem)

            def dg(g, base, sem):
                for b in range(8):
                    pltpu.make_async_copy(table.at[src_v.at[g * 8 + b]],
                                          rows_v.at[base + b], sem).wait()

            def isc(g, base, sem):
                for b in range(8):
                    pltpu.async_copy(rows_v.at[base + b],
                                     acc.at[dst_v.at[g * 8 + b]], sem,
                                     add=True)

            def dsc(g, base, sem):
                for b in range(8):
                    pltpu.make_async_copy(rows_v.at[base + b],
                                          acc.at[dst_v.at[g * 8 + b]],
                                          sem).wait()

            def phase(g, cur_base, sg_cur, ss_cur, oth_base, ss_oth):
                dsc(g - 1, oth_base, ss_oth)
                ig(g + 1, oth_base, sgb if cur_base == 0 else sga)
                dg(g, cur_base, sg_cur)
                isc(g, cur_base, ss_cur)

            def stage(si, carry):
                sb = row0 + si * _IB
                pltpu.sync_copy(srcr_hbm.at[pl.ds(sb, _IB)], src_v)
                pltpu.sync_copy(dstr_hbm.at[pl.ds(sb, _IB)], dst_v)

                ig(0, 0, sga)
                ig(1, 8, sgb)
                dg(0, 0, sga)
                isc(0, 0, ssa)

                def pair(t, c2):
                    phase(2 * t + 1, 8, sgb, ssb, 0, ssa)
                    phase(2 * t + 2, 0, sga, ssa, 8, ssb)
                    return c2
                lax.fori_loop(0, (_IB // 16) - 1, pair, 0)

                gl = _IB // 8 - 1
                dsc(gl - 1, 0, ssa)
                dg(gl, 8, sgb)
                isc(gl, 8, ssb)
                dsc(gl, 8, ssb)
                return carry
            lax.fori_loop(0, _ROWS_PER_TILE // _IB, stage, 0)
            plsc.subcore_barrier()
            pltpu.sync_copy(acc.at[pl.ds(s * _ZROWS, _ZROWS)],
                            out_ref.at[pl.ds(s * _ZROWS, _ZROWS)])

        for r in range(2):
            @pl.when(c == 0)
            def _(r=r):
                run(g_hbm.at[r], out_hbm.at[r])

            @pl.when(c == 1)
            def _(r=r):
                run(g_hbm.at[2 + r], out_hbm.at[2 + r])

    return k(g_q, zeros_blk, src_rows, dst_rows)


def _sc_deg(dst_rows):
    mesh = plsc.VectorSubcoreMesh(core_axis_name="c", subcore_axis_name="s")
    rows_per_core = _EROWS // 2
    rows_per_tile = rows_per_core // 16
    ib = 56

    @functools.partial(
        pl.kernel,
        out_type=jax.ShapeDtypeStruct((2, _NPAD), jnp.float32),
        mesh=mesh,
        scratch_types=[
            pltpu.VMEM((ib, 128), jnp.int32),
            pltpu.VMEM((128,), jnp.float32),
            pltpu.VMEM((128,), jnp.float32),
            pltpu.VMEM((_ZROWS,), jnp.float32),
            pltpu.VMEM_SHARED((_NPAD,), jnp.float32),
            pltpu.SemaphoreType.DMA,
        ],
        compiler_params=pltpu.CompilerParams(use_tc_tiling_on_sc=False),
    )
    def k(dstr_hbm, out_hbm, dst_v, ones_v, zbuf, wbuf, acc, sem):
        c = lax.axis_index("c")
        s = lax.axis_index("s")

        for i in range(8):
            ones_v[pl.ds(16 * i, 16)] = jnp.ones((16,), jnp.float32)
            zbuf[pl.ds(16 * i, 16)] = jnp.zeros((16,), jnp.float32)

        zbase = s * _ZROWS
        def zchunk(t, carry):
            pltpu.sync_copy(zbuf, acc.at[pl.ds(zbase + t * 128, 128)])
            return carry
        lax.fori_loop(0, 48, zchunk, 0)
        pltpu.sync_copy(zbuf.at[pl.ds(0, 112)],
                        acc.at[pl.ds(zbase + 48 * 128, 112)])
        plsc.subcore_barrier()

        row0 = c * rows_per_core + s * rows_per_tile

        def stage(si, carry):
            sb = row0 + si * ib
            pltpu.sync_copy(dstr_hbm.at[pl.ds(sb, ib)], dst_v)

            def one(j, c2):
                pltpu.sync_copy(ones_v, acc.at[dst_v.at[j]], add=True)
                return c2
            lax.fori_loop(0, ib, one, 0)
            return carry
        lax.fori_loop(0, rows_per_tile // ib, stage, 0)
        plsc.subcore_barrier()

        pltpu.sync_copy(acc.at[pl.ds(s * _ZROWS, _ZROWS)], wbuf)

        @pl.when(c == 0)
        def _():
            pltpu.sync_copy(wbuf, out_hbm.at[0].at[pl.ds(s * _ZROWS, _ZROWS)])

        @pl.when(c == 1)
        def _():
            pltpu.sync_copy(wbuf, out_hbm.at[1].at[pl.ds(s * _ZROWS, _ZROWS)])

    return k(dst_rows)


def _full(shape):
    return pl.BlockSpec(shape, lambda i: (0, 0))


def _rows(width):
    return pl.BlockSpec((_RB, width), lambda i: (i, 0))


def _rows4(n):
    return pl.BlockSpec((4, _RB, 8), lambda i: (0, i, 0))


def _tc_embed(x, We1, be1, We2, be2, Wg1, deg0, deg1):
    def body(x_r, w1_r, b1_r, w2_r, b2_r, wg_r, d0_r, d1_r, o_r):
        h = jnp.tanh(jnp.dot(x_r[...], w1_r[...],
                             preferred_element_type=jnp.float32) + b1_r[...])
        h = jnp.tanh(jnp.dot(h, w2_r[...],
                             preferred_element_type=jnp.float32) + b2_r[...])
        p = jnp.dot(h, wg_r[...], preferred_element_type=jnp.float32)
        dinv = lax.rsqrt(d0_r[...] + d1_r[...] + 1.0)
        g = p * dinv
        for q in range(4):
            o_r[q, :, :] = g[:, 8 * q:8 * (q + 1)]

    return pl.pallas_call(
        body,
        grid=(_N // _RB,),
        in_specs=[
            _rows(6), _full((6, 64)), _full((1, 64)), _full((64, 32)),
            _full((1, 32)), _full((32, 32)), _rows(1), _rows(1),
        ],
        out_specs=[_rows4(_N)],
        out_shape=[jax.ShapeDtypeStruct((4, _N, 8), jnp.float32)],
    )(x, We1, be1, We2, be2, Wg1, deg0, deg1)[0]


def _tc_mid(S, g, deg0, deg1, bg, Wg2):
    def body(s_r, g_r, d0_r, d1_r, bg_r, wg_r, o_r):
        dinv = lax.rsqrt(d0_r[...] + d1_r[...] + 1.0)
        p = None
        for q in range(4):
            hq = jnp.maximum((s_r[q, :, :] + g_r[q, :, :]) * dinv
                             + bg_r[...][:, 8 * q:8 * (q + 1)], 0.0)
            t = jnp.dot(hq, wg_r[...][8 * q:8 * (q + 1), :],
                        preferred_element_type=jnp.float32)
            p = t if p is None else p + t
        gout = p * dinv
        for q in range(4):
            o_r[q, :, :] = gout[:, 8 * q:8 * (q + 1)]

    return pl.pallas_call(
        body,
        grid=(_N // _RB,),
        in_specs=[
            _rows4(_NPAD), _rows4(_N), _rows(1), _rows(1),
            _full((1, 32)), _full((32, 32)),
        ],
        out_specs=[_rows4(_N)],
        out_shape=[jax.ShapeDtypeStruct((4, _N, 8), jnp.float32)],
    )(S, g, deg0, deg1, bg, Wg2)[0]


def _tc_head(S, g, deg0, deg1, bg, Wp1, bp1, Wp2, bp2):
    def body(s_r, g_r, d0_r, d1_r, bg_r, w1_r, b1_r, w2_r, b2_r, out_r):
        dinv = lax.rsqrt(d0_r[...] + d1_r[...] + 1.0)
        z = None
        for q in range(4):
            hq = jnp.maximum((s_r[q, :, :] + g_r[q, :, :]) * dinv
                             + bg_r[...][:, 8 * q:8 * (q + 1)], 0.0)
            t = jnp.dot(hq, w1_r[...][8 * q:8 * (q + 1), :],
                        preferred_element_type=jnp.float32)
            z = t if z is None else z + t
        z = jnp.tanh(z + b1_r[...])
        out_r[...] = jnp.tanh(jnp.dot(z, w2_r[...],
                                      preferred_element_type=jnp.float32)
                              + b2_r[...])

    return pl.pallas_call(
        body,
        grid=(_N // _RB,),
        in_specs=[
            _rows4(_NPAD), _rows4(_N), _rows(1), _rows(1),
            _full((1, 32)), _full((32, 32)), _full((1, 32)),
            _full((32, 1)), _full((1, 1)),
        ],
        out_specs=[_rows(1)],
        out_shape=[jax.ShapeDtypeStruct((_N, 1), jnp.float32)],
    )(S, g, deg0, deg1, bg, Wp1, bp1, Wp2, bp2)


def kernel(x, edge_index, We1, be1, We2, be2, Wg1, bg1, Wg2, bg2,
           Wp1, bp1, Wp2, bp2):
    src = edge_index[0]
    dst = edge_index[1]

    npad = _EROWS * 128 - _E
    fill = jnp.arange(npad, dtype=jnp.int32) % 64
    src_rows = jnp.concatenate([src, fill]).reshape(_EROWS, 128)
    dst_rows = jnp.concatenate([dst, _N + fill]).reshape(_EROWS, 128)

    degp = _sc_deg(dst_rows)
    deg0 = degp[0].reshape(_NPAD, 1)
    deg1 = degp[1].reshape(_NPAD, 1)

    be1r = be1.reshape(1, 64)
    be2r = be2.reshape(1, 32)
    bg1r = bg1.reshape(1, 32)
    bg2r = bg2.reshape(1, 32)
    bp1r = bp1.reshape(1, 32)
    bp2r = bp2.reshape(1, 1)

    zeros_blk = jnp.zeros((_ZROWS, 8), jnp.float32)

    g1 = _tc_embed(x, We1, be1r, We2, be2r, Wg1, deg0, deg1)
    S1 = _sc_scatter(g1, src_rows, dst_rows, zeros_blk)
    g2 = _tc_mid(S1, g1, deg0, deg1, bg1r, Wg2)
    S2 = _sc_scatter(g2, src_rows, dst_rows, zeros_blk)
    out, = _tc_head(S2, g2, deg0, deg1, bg2r, Wp1, bp1r, Wp2, bp2r)
    return out


# speedup vs baseline: 22.9411x; 1.0019x over previous
import functools

import jax
import jax.numpy as jnp
from jax import lax
from jax.experimental import pallas as pl
from jax.experimental.pallas import tpu as pltpu
from jax.experimental.pallas import tpu_sc as plsc

_N = 100000
_E = 1600000
_NPAD = 100096
_EROWS = 12544
_ROWS_PER_TILE = _EROWS // 16
_IB = 112
_WOUT = _N // 16
_ZROWS = _NPAD // 16

_RB = 2560


def _sc_scatter(g_q, src_rows, dst_rows, zeros_blk):
    mesh = plsc.VectorSubcoreMesh(core_axis_name="c", subcore_axis_name="s")

    @functools.partial(
        pl.kernel,
        out_type=jax.ShapeDtypeStruct((4, _NPAD, 8), jnp.float32),
        mesh=mesh,
        scratch_types=[
            pltpu.VMEM((_IB, 128), jnp.int32),
            pltpu.VMEM((_IB, 128), jnp.int32),
            pltpu.VMEM((16, 128, 8), jnp.float32),
            pltpu.VMEM_SHARED((_NPAD, 8), jnp.float32),
            pltpu.SemaphoreType.DMA,
            pltpu.SemaphoreType.DMA,
            pltpu.SemaphoreType.DMA,
            pltpu.SemaphoreType.DMA,
        ],
        compiler_params=pltpu.CompilerParams(use_tc_tiling_on_sc=False),
    )
    def k(g_hbm, zeros_hbm, srcr_hbm, dstr_hbm, out_hbm,
          src_v, dst_v, rows_v, acc, sga, sgb, ssa, ssb):
        c = lax.axis_index("c")
        s = lax.axis_index("s")

        def run(table, out_ref):
            pltpu.sync_copy(zeros_hbm, acc.at[pl.ds(s * _ZROWS, _ZROWS)])
            plsc.subcore_barrier()
            row0 = s * _ROWS_PER_TILE

            def ig(g, base, sem):
                for b in range(8):
                    pltpu.async_copy(table.at[src_v.at[g * 8 + b]],
                                     rows_v.at[base + b], sem)

            def dg(g, base, sem):
                for b in range(8):
                    pltpu.make_async_copy(table.at[src_v.at[g * 8 + b]],
                                          rows_v.at[base + b], sem).wait()

            def isc(g, base, sem):
                for b in range(8):
                    pltpu.async_copy(rows_v.at[base + b],
                                     acc.at[dst_v.at[g * 8 + b]], sem,
                                     add=True)

            def dsc(g, base, sem):
                for b in range(8):
                    pltpu.make_async_copy(rows_v.at[base + b],
                                          acc.at[dst_v.at[g * 8 + b]],
                                          sem).wait()

            def phase(g, cur_base, sg_cur, ss_cur, oth_base, ss_oth):
                dsc(g - 1, oth_base, ss_oth)
                ig(g + 1, oth_base, sgb if cur_base == 0 else sga)
                dg(g, cur_base, sg_cur)
                isc(g, cur_base, ss_cur)

            def stage(si, carry):
                sb = row0 + si * _IB
                pltpu.sync_copy(srcr_hbm.at[pl.ds(sb, _IB)], src_v)
                pltpu.sync_copy(dstr_hbm.at[pl.ds(sb, _IB)], dst_v)

                ig(0, 0, sga)
                ig(1, 8, sgb)
                dg(0, 0, sga)
                isc(0, 0, ssa)

                def pair(t, c2):
                    phase(2 * t + 1, 8, sgb, ssb, 0, ssa)
                    phase(2 * t + 2, 0, sga, ssa, 8, ssb)
                    return c2
                lax.fori_loop(0, (_IB // 16) - 1, pair, 0)

                gl = _IB // 8 - 1
                dsc(gl - 1, 0, ssa)
                dg(gl, 8, sgb)
                isc(gl, 8, ssb)
                dsc(gl, 8, ssb)
                return carry
            lax.fori_loop(0, _ROWS_PER_TILE // _IB, stage, 0)
            plsc.subcore_barrier()
            pltpu.sync_copy(acc.at[pl.ds(s * _ZROWS, _ZROWS)],
                            out_ref.at[pl.ds(s * _ZROWS, _ZROWS)])

        for r in range(2):
            @pl.when(c == 0)
            def _(r=r):
                run(g_hbm.at[r], out_hbm.at[r])

            @pl.when(c == 1)
            def _(r=r):
                run(g_hbm.at[2 + r], out_hbm.at[2 + r])

    return k(g_q, zeros_blk, src_rows, dst_rows)


def _sc_deg(dst_rows):
    mesh = plsc.VectorSubcoreMesh(core_axis_name="c", subcore_axis_name="s")
    rows_per_core = _EROWS // 2
    rows_per_tile = rows_per_core // 16
    ib = 56

    @functools.partial(
        pl.kernel,
        out_type=jax.ShapeDtypeStruct((2, _NPAD), jnp.float32),
        mesh=mesh,
        scratch_types=[
            pltpu.VMEM((ib, 128), jnp.int32),
            pltpu.VMEM((128,), jnp.float32),
            pltpu.VMEM((128,), jnp.float32),
            pltpu.VMEM((_ZROWS,), jnp.float32),
            pltpu.VMEM_SHARED((_NPAD,), jnp.float32),
            pltpu.SemaphoreType.DMA,
        ],
        compiler_params=pltpu.CompilerParams(use_tc_tiling_on_sc=False),
    )
    def k(dstr_hbm, out_hbm, dst_v, ones_v, zbuf, wbuf, acc, sem):
        c = lax.axis_index("c")
        s = lax.axis_index("s")

        for i in range(8):
            ones_v[pl.ds(16 * i, 16)] = jnp.ones((16,), jnp.float32)
            zbuf[pl.ds(16 * i, 16)] = jnp.zeros((16,), jnp.float32)

        zbase = s * _ZROWS
        def zchunk(t, carry):
            pltpu.sync_copy(zbuf, acc.at[pl.ds(zbase + t * 128, 128)])
            return carry
        lax.fori_loop(0, 48, zchunk, 0)
        pltpu.sync_copy(zbuf.at[pl.ds(0, 112)],
                        acc.at[pl.ds(zbase + 48 * 128, 112)])
        plsc.subcore_barrier()

        row0 = c * rows_per_core + s * rows_per_tile

        def stage(si, carry):
            sb = row0 + si * ib
            pltpu.sync_copy(dstr_hbm.at[pl.ds(sb, ib)], dst_v)

            def one(j, c2):
                pltpu.sync_copy(ones_v, acc.at[dst_v.at[j]], add=True)
                return c2
            lax.fori_loop(0, ib, one, 0)
            return carry
        lax.fori_loop(0, rows_per_tile // ib, stage, 0)
        plsc.subcore_barrier()

        pltpu.sync_copy(acc.at[pl.ds(s * _ZROWS, _ZROWS)], wbuf)

        @pl.when(c == 0)
        def _():
            pltpu.sync_copy(wbuf, out_hbm.at[0].at[pl.ds(s * _ZROWS, _ZROWS)])

        @pl.when(c == 1)
        def _():
            pltpu.sync_copy(wbuf, out_hbm.at[1].at[pl.ds(s * _ZROWS, _ZROWS)])

    return k(dst_rows)


def _full(shape):
    return pl.BlockSpec(shape, lambda i: (0, 0))


def _rows(width):
    return pl.BlockSpec((_RB, width), lambda i: (i, 0))


def _rows4(n):
    return pl.BlockSpec((4, _RB, 8), lambda i: (0, i, 0))


def _tc_embed(x, We1, be1, We2, be2, Wg1, deg0, deg1):
    def body(x_r, w1_r, b1_r, w2_r, b2_r, wg_r, d0_r, d1_r, o_r):
        h = jnp.tanh(jnp.dot(x_r[...], w1_r[...],
                             preferred_element_type=jnp.float32) + b1_r[...])
        h = jnp.tanh(jnp.dot(h, w2_r[...],
                             preferred_element_type=jnp.float32) + b2_r[...])
        p = jnp.dot(h, wg_r[...], preferred_element_type=jnp.float32)
        dinv = lax.rsqrt(d0_r[...] + d1_r[...] + 1.0)
        g = p * dinv
        for q in range(4):
            o_r[q, :, :] = g[:, 8 * q:8 * (q + 1)]

    return pl.pallas_call(
        body,
        grid=(-(-_N // _RB),),
        in_specs=[
            _rows(6), _full((6, 64)), _full((1, 64)), _full((64, 32)),
            _full((1, 32)), _full((32, 32)), _rows(1), _rows(1),
        ],
        out_specs=[_rows4(_N)],
        out_shape=[jax.ShapeDtypeStruct((4, _N, 8), jnp.float32)],
    )(x, We1, be1, We2, be2, Wg1, deg0, deg1)[0]


def _tc_mid(S, g, deg0, deg1, bg, Wg2):
    def body(s_r, g_r, d0_r, d1_r, bg_r, wg_r, o_r):
        dinv = lax.rsqrt(d0_r[...] + d1_r[...] + 1.0)
        p = None
        for q in range(4):
            hq = jnp.maximum((s_r[q, :, :] + g_r[q, :, :]) * dinv
                             + bg_r[...][:, 8 * q:8 * (q + 1)], 0.0)
            t = jnp.dot(hq, wg_r[...][8 * q:8 * (q + 1), :],
                        preferred_element_type=jnp.float32)
            p = t if p is None else p + t
        gout = p * dinv
        for q in range(4):
            o_r[q, :, :] = gout[:, 8 * q:8 * (q + 1)]

    return pl.pallas_call(
        body,
        grid=(-(-_N // _RB),),
        in_specs=[
            _rows4(_NPAD), _rows4(_N), _rows(1), _rows(1),
            _full((1, 32)), _full((32, 32)),
        ],
        out_specs=[_rows4(_N)],
        out_shape=[jax.ShapeDtypeStruct((4, _N, 8), jnp.float32)],
    )(S, g, deg0, deg1, bg, Wg2)[0]


def _tc_head(S, g, deg0, deg1, bg, Wp1, bp1, Wp2, bp2):
    def body(s_r, g_r, d0_r, d1_r, bg_r, w1_r, b1_r, w2_r, b2_r, out_r):
        dinv = lax.rsqrt(d0_r[...] + d1_r[...] + 1.0)
        z = None
        for q in range(4):
            hq = jnp.maximum((s_r[q, :, :] + g_r[q, :, :]) * dinv
                             + bg_r[...][:, 8 * q:8 * (q + 1)], 0.0)
            t = jnp.dot(hq, w1_r[...][8 * q:8 * (q + 1), :],
                        preferred_element_type=jnp.float32)
            z = t if z is None else z + t
        z = jnp.tanh(z + b1_r[...])
        out_r[...] = jnp.tanh(jnp.dot(z, w2_r[...],
                                      preferred_element_type=jnp.float32)
                              + b2_r[...])

    return pl.pallas_call(
        body,
        grid=(-(-_N // _RB),),
        in_specs=[
            _rows4(_NPAD), _rows4(_N), _rows(1), _rows(1),
            _full((1, 32)), _full((32, 32)), _full((1, 32)),
            _full((32, 1)), _full((1, 1)),
        ],
        out_specs=[_rows(1)],
        out_shape=[jax.ShapeDtypeStruct((_N, 1), jnp.float32)],
    )(S, g, deg0, deg1, bg, Wp1, bp1, Wp2, bp2)


def kernel(x, edge_index, We1, be1, We2, be2, Wg1, bg1, Wg2, bg2,
           Wp1, bp1, Wp2, bp2):
    src = edge_index[0]
    dst = edge_index[1]

    npad = _EROWS * 128 - _E
    fill = jnp.arange(npad, dtype=jnp.int32) % 64
    src_rows = jnp.concatenate([src, fill]).reshape(_EROWS, 128)
    dst_rows = jnp.concatenate([dst, _N + fill]).reshape(_EROWS, 128)

    degp = _sc_deg(dst_rows)
    deg0 = degp[0].reshape(_NPAD, 1)
    deg1 = degp[1].reshape(_NPAD, 1)

    be1r = be1.reshape(1, 64)
    be2r = be2.reshape(1, 32)
    bg1r = bg1.reshape(1, 32)
    bg2r = bg2.reshape(1, 32)
    bp1r = bp1.reshape(1, 32)
    bp2r = bp2.reshape(1, 1)

    zeros_blk = jnp.zeros((_ZROWS, 8), jnp.float32)

    g1 = _tc_embed(x, We1, be1r, We2, be2r, Wg1, deg0, deg1)
    S1 = _sc_scatter(g1, src_rows, dst_rows, zeros_blk)
    g2 = _tc_mid(S1, g1, deg0, deg1, bg1r, Wg2)
    S2 = _sc_scatter(g2, src_rows, dst_rows, zeros_blk)
    out, = _tc_head(S2, g2, deg0, deg1, bg2r, Wp1, bp1r, Wp2, bp2r)
    return out


# speedup vs baseline: 23.3910x; 1.0196x over previous
import functools

import jax
import jax.numpy as jnp
from jax import lax
from jax.experimental import pallas as pl
from jax.experimental.pallas import tpu as pltpu
from jax.experimental.pallas import tpu_sc as plsc

_N = 100000
_E = 1600000
_NPAD = 100096
_EROWS = 12544
_ROWS_PER_TILE = _EROWS // 16
_IB = 112
_WOUT = _N // 16
_ZROWS = _NPAD // 16

_RB = 2560


def _sc_scatter(g_q, src_rows, dst_rows, zeros_blk):
    mesh = plsc.VectorSubcoreMesh(core_axis_name="c", subcore_axis_name="s")

    @functools.partial(
        pl.kernel,
        out_type=jax.ShapeDtypeStruct((4, _NPAD, 8), jnp.float32),
        mesh=mesh,
        scratch_types=[
            pltpu.VMEM((_IB, 128), jnp.int32),
            pltpu.VMEM((_IB, 128), jnp.int32),
            pltpu.VMEM((24, 128, 8), jnp.float32),
            pltpu.VMEM_SHARED((_NPAD, 8), jnp.float32),
            pltpu.SemaphoreType.DMA,
            pltpu.SemaphoreType.DMA,
            pltpu.SemaphoreType.DMA,
            pltpu.SemaphoreType.DMA,
            pltpu.SemaphoreType.DMA,
            pltpu.SemaphoreType.DMA,
        ],
        compiler_params=pltpu.CompilerParams(use_tc_tiling_on_sc=False),
    )
    def k(g_hbm, zeros_hbm, srcr_hbm, dstr_hbm, out_hbm,
          src_v, dst_v, rows_v, acc, sga, sgb, sgc, ssa, ssb, ssc):
        c = lax.axis_index("c")
        s = lax.axis_index("s")

        def run(table, out_ref):
            pltpu.sync_copy(zeros_hbm, acc.at[pl.ds(s * _ZROWS, _ZROWS)])
            plsc.subcore_barrier()
            row0 = s * _ROWS_PER_TILE

            def ig(g, base, sem):
                for b in range(8):
                    pltpu.async_copy(table.at[src_v.at[g * 8 + b]],
                                     rows_v.at[base + b], sem)

            def dg(g, base, sem):
                for b in range(8):
                    pltpu.make_async_copy(table.at[src_v.at[g * 8 + b]],
                                          rows_v.at[base + b], sem).wait()

            def isc(g, base, sem):
                for b in range(8):
                    pltpu.async_copy(rows_v.at[base + b],
                                     acc.at[dst_v.at[g * 8 + b]], sem,
                                     add=True)

            def dsc(g, base, sem):
                for b in range(8):
                    pltpu.make_async_copy(rows_v.at[base + b],
                                          acc.at[dst_v.at[g * 8 + b]],
                                          sem).wait()

            sg = (sga, sgb, sgc)
            ss = (ssa, ssb, ssc)

            def phase(g, m):
                nm = (m + 2) % 3
                dsc(g - 1, 8 * nm, ss[nm])
                ig(g + 2, 8 * nm, sg[nm])
                dg(g, 8 * m, sg[m])
                isc(g, 8 * m, ss[m])

            def stage(si, carry):
                sb = row0 + si * _IB
                pltpu.sync_copy(srcr_hbm.at[pl.ds(sb, _IB)], src_v)
                pltpu.sync_copy(dstr_hbm.at[pl.ds(sb, _IB)], dst_v)

                ig(0, 0, sga)
                ig(1, 8, sgb)
                ig(2, 16, sgc)
                dg(0, 0, sga)
                isc(0, 0, ssa)

                def triple(t, c2):
                    phase(3 * t + 1, 1)
                    phase(3 * t + 2, 2)
                    phase(3 * t + 3, 0)
                    return c2
                lax.fori_loop(0, 3, triple, 0)

                phase(10, 1)
                phase(11, 2)
                dsc(11, 16, ssc)
                dg(12, 0, sga)
                isc(12, 0, ssa)
                dsc(12, 0, ssa)
                dg(13, 8, sgb)
                isc(13, 8, ssb)
                dsc(13, 8, ssb)
                return carry
            lax.fori_loop(0, _ROWS_PER_TILE // _IB, stage, 0)
            plsc.subcore_barrier()
            pltpu.sync_copy(acc.at[pl.ds(s * _ZROWS, _ZROWS)],
                            out_ref.at[pl.ds(s * _ZROWS, _ZROWS)])

        for r in range(2):
            @pl.when(c == 0)
            def _(r=r):
                run(g_hbm.at[r], out_hbm.at[r])

            @pl.when(c == 1)
            def _(r=r):
                run(g_hbm.at[2 + r], out_hbm.at[2 + r])

    return k(g_q, zeros_blk, src_rows, dst_rows)


def _sc_deg(dst_rows):
    mesh = plsc.VectorSubcoreMesh(core_axis_name="c", subcore_axis_name="s")
    rows_per_core = _EROWS // 2
    rows_per_tile = rows_per_core // 16
    ib = 56

    @functools.partial(
        pl.kernel,
        out_type=jax.ShapeDtypeStruct((2, _NPAD), jnp.float32),
        mesh=mesh,
        scratch_types=[
            pltpu.VMEM((ib, 128), jnp.int32),
            pltpu.VMEM((128,), jnp.float32),
            pltpu.VMEM((128,), jnp.float32),
            pltpu.VMEM((_ZROWS,), jnp.float32),
            pltpu.VMEM_SHARED((_NPAD,), jnp.float32),
            pltpu.SemaphoreType.DMA,
        ],
        compiler_params=pltpu.CompilerParams(use_tc_tiling_on_sc=False),
    )
    def k(dstr_hbm, out_hbm, dst_v, ones_v, zbuf, wbuf, acc, sem):
        c = lax.axis_index("c")
        s = lax.axis_index("s")

        for i in range(8):
            ones_v[pl.ds(16 * i, 16)] = jnp.ones((16,), jnp.float32)
            zbuf[pl.ds(16 * i, 16)] = jnp.zeros((16,), jnp.float32)

        zbase = s * _ZROWS
        def zchunk(t, carry):
            pltpu.sync_copy(zbuf, acc.at[pl.ds(zbase + t * 128, 128)])
            return carry
        lax.fori_loop(0, 48, zchunk, 0)
        pltpu.sync_copy(zbuf.at[pl.ds(0, 112)],
                        acc.at[pl.ds(zbase + 48 * 128, 112)])
        plsc.subcore_barrier()

        row0 = c * rows_per_core + s * rows_per_tile

        def stage(si, carry):
            sb = row0 + si * ib
            pltpu.sync_copy(dstr_hbm.at[pl.ds(sb, ib)], dst_v)

            def one(j, c2):
                pltpu.sync_copy(ones_v, acc.at[dst_v.at[j]], add=True)
                return c2
            lax.fori_loop(0, ib, one, 0)
            return carry
        lax.fori_loop(0, rows_per_tile // ib, stage, 0)
        plsc.subcore_barrier()

        pltpu.sync_copy(acc.at[pl.ds(s * _ZROWS, _ZROWS)], wbuf)

        @pl.when(c == 0)
        def _():
            pltpu.sync_copy(wbuf, out_hbm.at[0].at[pl.ds(s * _ZROWS, _ZROWS)])

        @pl.when(c == 1)
        def _():
            pltpu.sync_copy(wbuf, out_hbm.at[1].at[pl.ds(s * _ZROWS, _ZROWS)])

    return k(dst_rows)


def _full(shape):
    return pl.BlockSpec(shape, lambda i: (0, 0))


def _rows(width):
    return pl.BlockSpec((_RB, width), lambda i: (i, 0))


def _rows4(n):
    return pl.BlockSpec((4, _RB, 8), lambda i: (0, i, 0))


def _tc_embed(x, We1, be1, We2, be2, Wg1, deg0, deg1):
    def body(x_r, w1_r, b1_r, w2_r, b2_r, wg_r, d0_r, d1_r, o_r):
        h = jnp.tanh(jnp.dot(x_r[...], w1_r[...],
                             preferred_element_type=jnp.float32) + b1_r[...])
        h = jnp.tanh(jnp.dot(h, w2_r[...],
                             preferred_element_type=jnp.float32) + b2_r[...])
        p = jnp.dot(h, wg_r[...], preferred_element_type=jnp.float32)
        dinv = lax.rsqrt(d0_r[...] + d1_r[...] + 1.0)
        g = p * dinv
        for q in range(4):
            o_r[q, :, :] = g[:, 8 * q:8 * (q + 1)]

    return pl.pallas_call(
        body,
        grid=(-(-_N // _RB),),
        in_specs=[
            _rows(6), _full((6, 64)), _full((1, 64)), _full((64, 32)),
            _full((1, 32)), _full((32, 32)), _rows(1), _rows(1),
        ],
        out_specs=[_rows4(_N)],
        out_shape=[jax.ShapeDtypeStruct((4, _N, 8), jnp.float32)],
    )(x, We1, be1, We2, be2, Wg1, deg0, deg1)[0]


def _tc_mid(S, g, deg0, deg1, bg, Wg2):
    def body(s_r, g_r, d0_r, d1_r, bg_r, wg_r, o_r):
        dinv = lax.rsqrt(d0_r[...] + d1_r[...] + 1.0)
        p = None
        for q in range(4):
            hq = jnp.maximum((s_r[q, :, :] + g_r[q, :, :]) * dinv
                             + bg_r[...][:, 8 * q:8 * (q + 1)], 0.0)
            t = jnp.dot(hq, wg_r[...][8 * q:8 * (q + 1), :],
                        preferred_element_type=jnp.float32)
            p = t if p is None else p + t
        gout = p * dinv
        for q in range(4):
            o_r[q, :, :] = gout[:, 8 * q:8 * (q + 1)]

    return pl.pallas_call(
        body,
        grid=(-(-_N // _RB),),
        in_specs=[
            _rows4(_NPAD), _rows4(_N), _rows(1), _rows(1),
            _full((1, 32)), _full((32, 32)),
        ],
        out_specs=[_rows4(_N)],
        out_shape=[jax.ShapeDtypeStruct((4, _N, 8), jnp.float32)],
    )(S, g, deg0, deg1, bg, Wg2)[0]


def _tc_head(S, g, deg0, deg1, bg, Wp1, bp1, Wp2, bp2):
    def body(s_r, g_r, d0_r, d1_r, bg_r, w1_r, b1_r, w2_r, b2_r, out_r):
        dinv = lax.rsqrt(d0_r[...] + d1_r[...] + 1.0)
        z = None
        for q in range(4):
            hq = jnp.maximum((s_r[q, :, :] + g_r[q, :, :]) * dinv
                             + bg_r[...][:, 8 * q:8 * (q + 1)], 0.0)
            t = jnp.dot(hq, w1_r[...][8 * q:8 * (q + 1), :],
                        preferred_element_type=jnp.float32)
            z = t if z is None else z + t
        z = jnp.tanh(z + b1_r[...])
        out_r[...] = jnp.tanh(jnp.dot(z, w2_r[...],
                                      preferred_element_type=jnp.float32)
                              + b2_r[...])

    return pl.pallas_call(
        body,
        grid=(-(-_N // _RB),),
        in_specs=[
            _rows4(_NPAD), _rows4(_N), _rows(1), _rows(1),
            _full((1, 32)), _full((32, 32)), _full((1, 32)),
            _full((32, 1)), _full((1, 1)),
        ],
        out_specs=[_rows(1)],
        out_shape=[jax.ShapeDtypeStruct((_N, 1), jnp.float32)],
    )(S, g, deg0, deg1, bg, Wp1, bp1, Wp2, bp2)


def kernel(x, edge_index, We1, be1, We2, be2, Wg1, bg1, Wg2, bg2,
           Wp1, bp1, Wp2, bp2):
    src = edge_index[0]
    dst = edge_index[1]

    npad = _EROWS * 128 - _E
    fill = jnp.arange(npad, dtype=jnp.int32) % 64
    src_rows = jnp.concatenate([src, fill]).reshape(_EROWS, 128)
    dst_rows = jnp.concatenate([dst, _N + fill]).reshape(_EROWS, 128)

    degp = _sc_deg(dst_rows)
    deg0 = degp[0].reshape(_NPAD, 1)
    deg1 = degp[1].reshape(_NPAD, 1)

    be1r = be1.reshape(1, 64)
    be2r = be2.reshape(1, 32)
    bg1r = bg1.reshape(1, 32)
    bg2r = bg2.reshape(1, 32)
    bp1r = bp1.reshape(1, 32)
    bp2r = bp2.reshape(1, 1)

    zeros_blk = jnp.zeros((_ZROWS, 8), jnp.float32)

    g1 = _tc_embed(x, We1, be1r, We2, be2r, Wg1, deg0, deg1)
    S1 = _sc_scatter(g1, src_rows, dst_rows, zeros_blk)
    g2 = _tc_mid(S1, g1, deg0, deg1, bg1r, Wg2)
    S2 = _sc_scatter(g2, src_rows, dst_rows, zeros_blk)
    out, = _tc_head(S2, g2, deg0, deg1, bg2r, Wp1, bp1r, Wp2, bp2r)
    return out


# speedup vs baseline: 25.7228x; 1.0997x over previous
import functools

import jax
import jax.numpy as jnp
from jax import lax
from jax.experimental import pallas as pl
from jax.experimental.pallas import tpu as pltpu
from jax.experimental.pallas import tpu_sc as plsc

_N = 100000
_E = 1600000
_NPAD = 100096
_EROWS = 12544
_ROWS_PER_TILE = _EROWS // 16
_IB = 112
_WOUT = _N // 16
_ZROWS = _NPAD // 16

_RB = 2560


def _sc_scatter(g_pair, src_rows, dst_rows, zeros_blk):
    mesh = plsc.VectorSubcoreMesh(core_axis_name="c", subcore_axis_name="s")

    @functools.partial(
        pl.kernel,
        out_type=jax.ShapeDtypeStruct((2, _NPAD, 8), jnp.float32),
        mesh=mesh,
        scratch_types=[
            pltpu.VMEM((_IB, 128), jnp.int32),
            pltpu.VMEM((_IB, 128), jnp.int32),
            pltpu.VMEM((24, 128, 8), jnp.float32),
            pltpu.VMEM_SHARED((_NPAD, 8), jnp.float32),
            pltpu.SemaphoreType.DMA,
            pltpu.SemaphoreType.DMA,
            pltpu.SemaphoreType.DMA,
            pltpu.SemaphoreType.DMA,
            pltpu.SemaphoreType.DMA,
            pltpu.SemaphoreType.DMA,
        ],
        compiler_params=pltpu.CompilerParams(use_tc_tiling_on_sc=False),
    )
    def k(g_hbm, zeros_hbm, srcr_hbm, dstr_hbm, out_hbm,
          src_v, dst_v, rows_v, acc, sga, sgb, sgc, ssa, ssb, ssc):
        c = lax.axis_index("c")
        s = lax.axis_index("s")

        def run(table, out_ref):
            pltpu.sync_copy(zeros_hbm, acc.at[pl.ds(s * _ZROWS, _ZROWS)])
            plsc.subcore_barrier()
            row0 = s * _ROWS_PER_TILE

            def ig(g, base, sem):
                for b in range(8):
                    pltpu.async_copy(table.at[src_v.at[g * 8 + b]],
                                     rows_v.at[base + b], sem)

            def dg(g, base, sem):
                for b in range(8):
                    pltpu.make_async_copy(table.at[src_v.at[g * 8 + b]],
                                          rows_v.at[base + b], sem).wait()

            def isc(g, base, sem):
                for b in range(8):
                    pltpu.async_copy(rows_v.at[base + b],
                                     acc.at[dst_v.at[g * 8 + b]], sem,
                                     add=True)

            def dsc(g, base, sem):
                for b in range(8):
                    pltpu.make_async_copy(rows_v.at[base + b],
                                          acc.at[dst_v.at[g * 8 + b]],
                                          sem).wait()

            sg = (sga, sgb, sgc)
            ss = (ssa, ssb, ssc)

            def phase(g, m):
                nm = (m + 2) % 3
                dsc(g - 1, 8 * nm, ss[nm])
                ig(g + 2, 8 * nm, sg[nm])
                dg(g, 8 * m, sg[m])
                isc(g, 8 * m, ss[m])

            def stage(si, carry):
                sb = row0 + si * _IB
                pltpu.sync_copy(srcr_hbm.at[pl.ds(sb, _IB)], src_v)
                pltpu.sync_copy(dstr_hbm.at[pl.ds(sb, _IB)], dst_v)

                ig(0, 0, sga)
                ig(1, 8, sgb)
                ig(2, 16, sgc)
                dg(0, 0, sga)
                isc(0, 0, ssa)

                def triple(t, c2):
                    phase(3 * t + 1, 1)
                    phase(3 * t + 2, 2)
                    phase(3 * t + 3, 0)
                    return c2
                lax.fori_loop(0, 3, triple, 0)

                phase(10, 1)
                phase(11, 2)
                dsc(11, 16, ssc)
                dg(12, 0, sga)
                isc(12, 0, ssa)
                dsc(12, 0, ssa)
                dg(13, 8, sgb)
                isc(13, 8, ssb)
                dsc(13, 8, ssb)
                return carry
            lax.fori_loop(0, _ROWS_PER_TILE // _IB, stage, 0)
            plsc.subcore_barrier()
            pltpu.sync_copy(acc.at[pl.ds(s * _ZROWS, _ZROWS)],
                            out_ref.at[pl.ds(s * _ZROWS, _ZROWS)])

        @pl.when(c == 0)
        def _():
            run(g_hbm.at[0], out_hbm.at[0])

        @pl.when(c == 1)
        def _():
            run(g_hbm.at[1], out_hbm.at[1])

    return k(g_pair, zeros_blk, src_rows, dst_rows)


def _sc_deg(dst_rows):
    mesh = plsc.VectorSubcoreMesh(core_axis_name="c", subcore_axis_name="s")
    rows_per_core = _EROWS // 2
    rows_per_tile = rows_per_core // 16
    ib = 56

    @functools.partial(
        pl.kernel,
        out_type=jax.ShapeDtypeStruct((2, _NPAD), jnp.float32),
        mesh=mesh,
        scratch_types=[
            pltpu.VMEM((ib, 128), jnp.int32),
            pltpu.VMEM((128,), jnp.float32),
            pltpu.VMEM((128,), jnp.float32),
            pltpu.VMEM((_ZROWS,), jnp.float32),
            pltpu.VMEM_SHARED((_NPAD,), jnp.float32),
            pltpu.SemaphoreType.DMA,
        ],
        compiler_params=pltpu.CompilerParams(use_tc_tiling_on_sc=False),
    )
    def k(dstr_hbm, out_hbm, dst_v, ones_v, zbuf, wbuf, acc, sem):
        c = lax.axis_index("c")
        s = lax.axis_index("s")

        for i in range(8):
            ones_v[pl.ds(16 * i, 16)] = jnp.ones((16,), jnp.float32)
            zbuf[pl.ds(16 * i, 16)] = jnp.zeros((16,), jnp.float32)

        zbase = s * _ZROWS
        def zchunk(t, carry):
            pltpu.sync_copy(zbuf, acc.at[pl.ds(zbase + t * 128, 128)])
            return carry
        lax.fori_loop(0, 48, zchunk, 0)
        pltpu.sync_copy(zbuf.at[pl.ds(0, 112)],
                        acc.at[pl.ds(zbase + 48 * 128, 112)])
        plsc.subcore_barrier()

        row0 = c * rows_per_core + s * rows_per_tile

        def stage(si, carry):
            sb = row0 + si * ib
            pltpu.sync_copy(dstr_hbm.at[pl.ds(sb, ib)], dst_v)

            def one(j, c2):
                pltpu.sync_copy(ones_v, acc.at[dst_v.at[j]], add=True)
                return c2
            lax.fori_loop(0, ib, one, 0)
            return carry
        lax.fori_loop(0, rows_per_tile // ib, stage, 0)
        plsc.subcore_barrier()

        pltpu.sync_copy(acc.at[pl.ds(s * _ZROWS, _ZROWS)], wbuf)

        @pl.when(c == 0)
        def _():
            pltpu.sync_copy(wbuf, out_hbm.at[0].at[pl.ds(s * _ZROWS, _ZROWS)])

        @pl.when(c == 1)
        def _():
            pltpu.sync_copy(wbuf, out_hbm.at[1].at[pl.ds(s * _ZROWS, _ZROWS)])

    return k(dst_rows)


def _full(shape):
    return pl.BlockSpec(shape, lambda i: (0, 0))


def _rows(width):
    return pl.BlockSpec((_RB, width), lambda i: (i, 0))


def _rows2(n):
    return pl.BlockSpec((2, _RB, 8), lambda i: (0, i, 0))


def _tc_embed(x, We1, be1, We2, be2, Wg1, deg0, deg1):
    def body(x_r, w1_r, b1_r, w2_r, b2_r, wg_r, d0_r, d1_r, oa_r, ob_r):
        h = jnp.tanh(jnp.dot(x_r[...], w1_r[...],
                             preferred_element_type=jnp.float32) + b1_r[...])
        h = jnp.tanh(jnp.dot(h, w2_r[...],
                             preferred_element_type=jnp.float32) + b2_r[...])
        p = jnp.dot(h, wg_r[...], preferred_element_type=jnp.float32)
        dinv = lax.rsqrt(d0_r[...] + d1_r[...] + 1.0)
        g = p * dinv
        for q in range(2):
            oa_r[q, :, :] = g[:, 8 * q:8 * (q + 1)]
            ob_r[q, :, :] = g[:, 16 + 8 * q:16 + 8 * (q + 1)]

    return pl.pallas_call(
        body,
        grid=(-(-_N // _RB),),
        in_specs=[
            _rows(6), _full((6, 64)), _full((1, 64)), _full((64, 32)),
            _full((1, 32)), _full((32, 32)), _rows(1), _rows(1),
        ],
        out_specs=[_rows2(_N)] * 2,
        out_shape=[jax.ShapeDtypeStruct((2, _N, 8), jnp.float32)] * 2,
    )(x, We1, be1, We2, be2, Wg1, deg0, deg1)


def _tc_mid(Sa, Sb, ga, gb, deg0, deg1, bg, Wg2):
    def body(sa_r, sb_r, ga_r, gb_r, d0_r, d1_r, bg_r, wg_r, oa_r, ob_r):
        dinv = lax.rsqrt(d0_r[...] + d1_r[...] + 1.0)
        p = None
        for q in range(4):
            s_q = (sa_r, sb_r)[q // 2][q % 2, :, :]
            g_q = (ga_r, gb_r)[q // 2][q % 2, :, :]
            hq = jnp.maximum((s_q + g_q) * dinv
                             + bg_r[...][:, 8 * q:8 * (q + 1)], 0.0)
            t = jnp.dot(hq, wg_r[...][8 * q:8 * (q + 1), :],
                        preferred_element_type=jnp.float32)
            p = t if p is None else p + t
        gout = p * dinv
        for q in range(2):
            oa_r[q, :, :] = gout[:, 8 * q:8 * (q + 1)]
            ob_r[q, :, :] = gout[:, 16 + 8 * q:16 + 8 * (q + 1)]

    return pl.pallas_call(
        body,
        grid=(-(-_N // _RB),),
        in_specs=[
            _rows2(_NPAD), _rows2(_NPAD), _rows2(_N), _rows2(_N),
            _rows(1), _rows(1), _full((1, 32)), _full((32, 32)),
        ],
        out_specs=[_rows2(_N)] * 2,
        out_shape=[jax.ShapeDtypeStruct((2, _N, 8), jnp.float32)] * 2,
    )(Sa, Sb, ga, gb, deg0, deg1, bg, Wg2)


def _tc_head(Sa, Sb, ga, gb, deg0, deg1, bg, Wp1, bp1, Wp2, bp2):
    def body(sa_r, sb_r, ga_r, gb_r, d0_r, d1_r, bg_r, w1_r, b1_r,
             w2_r, b2_r, out_r):
        dinv = lax.rsqrt(d0_r[...] + d1_r[...] + 1.0)
        z = None
        for q in range(4):
            s_q = (sa_r, sb_r)[q // 2][q % 2, :, :]
            g_q = (ga_r, gb_r)[q // 2][q % 2, :, :]
            hq = jnp.maximum((s_q + g_q) * dinv
                             + bg_r[...][:, 8 * q:8 * (q + 1)], 0.0)
            t = jnp.dot(hq, w1_r[...][8 * q:8 * (q + 1), :],
                        preferred_element_type=jnp.float32)
            z = t if z is None else z + t
        z = jnp.tanh(z + b1_r[...])
        out_r[...] = jnp.tanh(jnp.dot(z, w2_r[...],
                                      preferred_element_type=jnp.float32)
                              + b2_r[...])

    return pl.pallas_call(
        body,
        grid=(-(-_N // _RB),),
        in_specs=[
            _rows2(_NPAD), _rows2(_NPAD), _rows2(_N), _rows2(_N),
            _rows(1), _rows(1), _full((1, 32)), _full((32, 32)),
            _full((1, 32)), _full((32, 1)), _full((1, 1)),
        ],
        out_specs=[_rows(1)],
        out_shape=[jax.ShapeDtypeStruct((_N, 1), jnp.float32)],
    )(Sa, Sb, ga, gb, deg0, deg1, bg, Wp1, bp1, Wp2, bp2)


def kernel(x, edge_index, We1, be1, We2, be2, Wg1, bg1, Wg2, bg2,
           Wp1, bp1, Wp2, bp2):
    src = edge_index[0]
    dst = edge_index[1]

    npad = _EROWS * 128 - _E
    fill = jnp.arange(npad, dtype=jnp.int32) % 64
    src_rows = jnp.concatenate([src, fill]).reshape(_EROWS, 128)
    dst_rows = jnp.concatenate([dst, _N + fill]).reshape(_EROWS, 128)

    degp = _sc_deg(dst_rows)
    deg0 = degp[0].reshape(_NPAD, 1)
    deg1 = degp[1].reshape(_NPAD, 1)

    be1r = be1.reshape(1, 64)
    be2r = be2.reshape(1, 32)
    bg1r = bg1.reshape(1, 32)
    bg2r = bg2.reshape(1, 32)
    bp1r = bp1.reshape(1, 32)
    bp2r = bp2.reshape(1, 1)

    zeros_blk = jnp.zeros((_ZROWS, 8), jnp.float32)

    g1a, g1b = _tc_embed(x, We1, be1r, We2, be2r, Wg1, deg0, deg1)
    S1a = _sc_scatter(g1a, src_rows, dst_rows, zeros_blk)
    S1b = _sc_scatter(g1b, src_rows, dst_rows, zeros_blk)
    g2a, g2b = _tc_mid(S1a, S1b, g1a, g1b, deg0, deg1, bg1r, Wg2)
    S2a = _sc_scatter(g2a, src_rows, dst_rows, zeros_blk)
    S2b = _sc_scatter(g2b, src_rows, dst_rows, zeros_blk)
    out, = _tc_head(S2a, S2b, g2a, g2b, deg0, deg1, bg2r,
                    Wp1, bp1r, Wp2, bp2r)
    return out


# speedup vs baseline: 26.1384x; 1.0162x over previous
import functools

import jax
import jax.numpy as jnp
from jax import lax
from jax.experimental import pallas as pl
from jax.experimental.pallas import tpu as pltpu
from jax.experimental.pallas import tpu_sc as plsc

_N = 100000
_E = 1600000
_NPAD = 100096
_EROWS = 12544
_ROWS_PER_TILE = _EROWS // 16
_IB = 112
_WOUT = _N // 16
_ZROWS = _NPAD // 16

_RB = 3584


def _sc_scatter(g_pair, e_rows, zeros_blk):
    mesh = plsc.VectorSubcoreMesh(core_axis_name="c", subcore_axis_name="s")

    @functools.partial(
        pl.kernel,
        out_type=jax.ShapeDtypeStruct((2, _NPAD, 8), jnp.float32),
        mesh=mesh,
        scratch_types=[
            pltpu.VMEM((_IB, 128), jnp.int32),
            pltpu.VMEM((_IB, 128), jnp.int32),
            pltpu.VMEM((24, 128, 8), jnp.float32),
            pltpu.VMEM_SHARED((_NPAD, 8), jnp.float32),
            pltpu.SemaphoreType.DMA,
            pltpu.SemaphoreType.DMA,
            pltpu.SemaphoreType.DMA,
            pltpu.SemaphoreType.DMA,
            pltpu.SemaphoreType.DMA,
            pltpu.SemaphoreType.DMA,
        ],
        compiler_params=pltpu.CompilerParams(use_tc_tiling_on_sc=False),
    )
    def k(g_hbm, zeros_hbm, er_hbm, out_hbm,
          src_v, dst_v, rows_v, acc, sga, sgb, sgc, ssa, ssb, ssc):
        c = lax.axis_index("c")
        s = lax.axis_index("s")

        def run(table, out_ref):
            pltpu.sync_copy(zeros_hbm, acc.at[pl.ds(s * _ZROWS, _ZROWS)])
            plsc.subcore_barrier()
            row0 = s * _ROWS_PER_TILE

            def ig(g, base, sem):
                for b in range(8):
                    pltpu.async_copy(table.at[src_v.at[g * 8 + b]],
                                     rows_v.at[base + b], sem)

            def dg(g, base, sem):
                for b in range(8):
                    pltpu.make_async_copy(table.at[src_v.at[g * 8 + b]],
                                          rows_v.at[base + b], sem).wait()

            def isc(g, base, sem):
                for b in range(8):
                    pltpu.async_copy(rows_v.at[base + b],
                                     acc.at[dst_v.at[g * 8 + b]], sem,
                                     add=True)

            def dsc(g, base, sem):
                for b in range(8):
                    pltpu.make_async_copy(rows_v.at[base + b],
                                          acc.at[dst_v.at[g * 8 + b]],
                                          sem).wait()

            sg = (sga, sgb, sgc)
            ss = (ssa, ssb, ssc)

            def phase(g, m):
                nm = (m + 2) % 3
                dsc(g - 1, 8 * nm, ss[nm])
                ig(g + 2, 8 * nm, sg[nm])
                dg(g, 8 * m, sg[m])
                isc(g, 8 * m, ss[m])

            def stage(si, carry):
                sb = row0 + si * _IB
                pltpu.sync_copy(er_hbm.at[0].at[pl.ds(sb, _IB)], src_v)
                pltpu.sync_copy(er_hbm.at[1].at[pl.ds(sb, _IB)], dst_v)

                ig(0, 0, sga)
                ig(1, 8, sgb)
                ig(2, 16, sgc)
                dg(0, 0, sga)
                isc(0, 0, ssa)

                def triple(t, c2):
                    phase(3 * t + 1, 1)
                    phase(3 * t + 2, 2)
                    phase(3 * t + 3, 0)
                    return c2
                lax.fori_loop(0, 3, triple, 0)

                phase(10, 1)
                phase(11, 2)
                dsc(11, 16, ssc)
                dg(12, 0, sga)
                isc(12, 0, ssa)
                dsc(12, 0, ssa)
                dg(13, 8, sgb)
                isc(13, 8, ssb)
                dsc(13, 8, ssb)
                return carry
            lax.fori_loop(0, _ROWS_PER_TILE // _IB, stage, 0)
            plsc.subcore_barrier()
            pltpu.sync_copy(acc.at[pl.ds(s * _ZROWS, _ZROWS)],
                            out_ref.at[pl.ds(s * _ZROWS, _ZROWS)])

        @pl.when(c == 0)
        def _():
            run(g_hbm.at[0], out_hbm.at[0])

        @pl.when(c == 1)
        def _():
            run(g_hbm.at[1], out_hbm.at[1])

    return k(g_pair, zeros_blk, e_rows)


def _sc_deg(e_rows):
    mesh = plsc.VectorSubcoreMesh(core_axis_name="c", subcore_axis_name="s")
    rows_per_core = _EROWS // 2
    rows_per_tile = rows_per_core // 16
    ib = 56

    @functools.partial(
        pl.kernel,
        out_type=jax.ShapeDtypeStruct((2, _NPAD), jnp.float32),
        mesh=mesh,
        scratch_types=[
            pltpu.VMEM((ib, 128), jnp.int32),
            pltpu.VMEM((128,), jnp.float32),
            pltpu.VMEM((128,), jnp.float32),
            pltpu.VMEM((_ZROWS,), jnp.float32),
            pltpu.VMEM_SHARED((_NPAD,), jnp.float32),
            pltpu.SemaphoreType.DMA,
        ],
        compiler_params=pltpu.CompilerParams(use_tc_tiling_on_sc=False),
    )
    def k(er_hbm, out_hbm, dst_v, ones_v, zbuf, wbuf, acc, sem):
        c = lax.axis_index("c")
        s = lax.axis_index("s")

        for i in range(8):
            ones_v[pl.ds(16 * i, 16)] = jnp.ones((16,), jnp.float32)
            zbuf[pl.ds(16 * i, 16)] = jnp.zeros((16,), jnp.float32)

        zbase = s * _ZROWS
        def zchunk(t, carry):
            pltpu.sync_copy(zbuf, acc.at[pl.ds(zbase + t * 128, 128)])
            return carry
        lax.fori_loop(0, 48, zchunk, 0)
        pltpu.sync_copy(zbuf.at[pl.ds(0, 112)],
                        acc.at[pl.ds(zbase + 48 * 128, 112)])
        plsc.subcore_barrier()

        row0 = c * rows_per_core + s * rows_per_tile

        def stage(si, carry):
            sb = row0 + si * ib
            pltpu.sync_copy(er_hbm.at[1].at[pl.ds(sb, ib)], dst_v)

            def one(j, c2):
                pltpu.sync_copy(ones_v, acc.at[dst_v.at[j]], add=True)
                return c2
            lax.fori_loop(0, ib, one, 0)
            return carry
        lax.fori_loop(0, rows_per_tile // ib, stage, 0)
        plsc.subcore_barrier()

        pltpu.sync_copy(acc.at[pl.ds(s * _ZROWS, _ZROWS)], wbuf)

        @pl.when(c == 0)
        def _():
            pltpu.sync_copy(wbuf, out_hbm.at[0].at[pl.ds(s * _ZROWS, _ZROWS)])

        @pl.when(c == 1)
        def _():
            pltpu.sync_copy(wbuf, out_hbm.at[1].at[pl.ds(s * _ZROWS, _ZROWS)])

    return k(e_rows)


def _full(shape):
    return pl.BlockSpec(shape, lambda i: (0, 0))


def _rows(width):
    return pl.BlockSpec((_RB, width), lambda i: (i, 0))


def _rows2(n):
    return pl.BlockSpec((2, _RB, 8), lambda i: (0, i, 0))


def _tc_embed(x, We1, be1, We2, be2, Wg1, deg0, deg1):
    def body(x_r, w1_r, b1_r, w2_r, b2_r, wg_r, d0_r, d1_r, oa_r, ob_r):
        h = jnp.tanh(jnp.dot(x_r[...], w1_r[...],
                             preferred_element_type=jnp.float32) + b1_r[...])
        h = jnp.tanh(jnp.dot(h, w2_r[...],
                             preferred_element_type=jnp.float32) + b2_r[...])
        p = jnp.dot(h, wg_r[...], preferred_element_type=jnp.float32)
        dinv = lax.rsqrt(d0_r[...] + d1_r[...] + 1.0)
        g = p * dinv
        for q in range(2):
            oa_r[q, :, :] = g[:, 8 * q:8 * (q + 1)]
            ob_r[q, :, :] = g[:, 16 + 8 * q:16 + 8 * (q + 1)]

    return pl.pallas_call(
        body,
        grid=(-(-_N // _RB),),
        in_specs=[
            _rows(6), _full((6, 64)), _full((1, 64)), _full((64, 32)),
            _full((1, 32)), _full((32, 32)), _rows(1), _rows(1),
        ],
        out_specs=[_rows2(_N)] * 2,
        out_shape=[jax.ShapeDtypeStruct((2, _N, 8), jnp.float32)] * 2,
    )(x, We1, be1, We2, be2, Wg1, deg0, deg1)


def _tc_mid(Sa, Sb, ga, gb, deg0, deg1, bg, Wg2):
    def body(sa_r, sb_r, ga_r, gb_r, d0_r, d1_r, bg_r, wg_r, oa_r, ob_r):
        dinv = lax.rsqrt(d0_r[...] + d1_r[...] + 1.0)
        p = None
        for q in range(4):
            s_q = (sa_r, sb_r)[q // 2][q % 2, :, :]
            g_q = (ga_r, gb_r)[q // 2][q % 2, :, :]
            hq = jnp.maximum((s_q + g_q) * dinv
                             + bg_r[...][:, 8 * q:8 * (q + 1)], 0.0)
            t = jnp.dot(hq, wg_r[...][8 * q:8 * (q + 1), :],
                        preferred_element_type=jnp.float32)
            p = t if p is None else p + t
        gout = p * dinv
        for q in range(2):
            oa_r[q, :, :] = gout[:, 8 * q:8 * (q + 1)]
            ob_r[q, :, :] = gout[:, 16 + 8 * q:16 + 8 * (q + 1)]

    return pl.pallas_call(
        body,
        grid=(-(-_N // _RB),),
        in_specs=[
            _rows2(_NPAD), _rows2(_NPAD), _rows2(_N), _rows2(_N),
            _rows(1), _rows(1), _full((1, 32)), _full((32, 32)),
        ],
        out_specs=[_rows2(_N)] * 2,
        out_shape=[jax.ShapeDtypeStruct((2, _N, 8), jnp.float32)] * 2,
    )(Sa, Sb, ga, gb, deg0, deg1, bg, Wg2)


def _tc_head(Sa, Sb, ga, gb, deg0, deg1, bg, Wp1, bp1, Wp2, bp2):
    def body(sa_r, sb_r, ga_r, gb_r, d0_r, d1_r, bg_r, w1_r, b1_r,
             w2_r, b2_r, out_r):
        dinv = lax.rsqrt(d0_r[...] + d1_r[...] + 1.0)
        z = None
        for q in range(4):
            s_q = (sa_r, sb_r)[q // 2][q % 2, :, :]
            g_q = (ga_r, gb_r)[q // 2][q % 2, :, :]
            hq = jnp.maximum((s_q + g_q) * dinv
                             + bg_r[...][:, 8 * q:8 * (q + 1)], 0.0)
            t = jnp.dot(hq, w1_r[...][8 * q:8 * (q + 1), :],
                        preferred_element_type=jnp.float32)
            z = t if z is None else z + t
        z = jnp.tanh(z + b1_r[...])
        out_r[...] = jnp.tanh(jnp.dot(z, w2_r[...],
                                      preferred_element_type=jnp.float32)
                              + b2_r[...])

    return pl.pallas_call(
        body,
        grid=(-(-_N // _RB),),
        in_specs=[
            _rows2(_NPAD), _rows2(_NPAD), _rows2(_N), _rows2(_N),
            _rows(1), _rows(1), _full((1, 32)), _full((32, 32)),
            _full((1, 32)), _full((32, 1)), _full((1, 1)),
        ],
        out_specs=[_rows(1)],
        out_shape=[jax.ShapeDtypeStruct((_N, 1), jnp.float32)],
    )(Sa, Sb, ga, gb, deg0, deg1, bg, Wp1, bp1, Wp2, bp2)


def kernel(x, edge_index, We1, be1, We2, be2, Wg1, bg1, Wg2, bg2,
           Wp1, bp1, Wp2, bp2):
    npad = _EROWS * 128 - _E
    fill = jnp.arange(npad, dtype=jnp.int32) % 64
    pad2 = jnp.stack([fill, _N + fill])
    e_rows = jnp.concatenate([edge_index, pad2], axis=1).reshape(
        2, _EROWS, 128)

    degp = _sc_deg(e_rows)
    deg0 = degp[0].reshape(_NPAD, 1)
    deg1 = degp[1].reshape(_NPAD, 1)

    be1r = be1.reshape(1, 64)
    be2r = be2.reshape(1, 32)
    bg1r = bg1.reshape(1, 32)
    bg2r = bg2.reshape(1, 32)
    bp1r = bp1.reshape(1, 32)
    bp2r = bp2.reshape(1, 1)

    zeros_blk = jnp.zeros((_ZROWS, 8), jnp.float32)

    g1a, g1b = _tc_embed(x, We1, be1r, We2, be2r, Wg1, deg0, deg1)
    S1a = _sc_scatter(g1a, e_rows, zeros_blk)
    S1b = _sc_scatter(g1b, e_rows, zeros_blk)
    g2a, g2b = _tc_mid(S1a, S1b, g1a, g1b, deg0, deg1, bg1r, Wg2)
    S2a = _sc_scatter(g2a, e_rows, zeros_blk)
    S2b = _sc_scatter(g2b, e_rows, zeros_blk)
    out, = _tc_head(S2a, S2b, g2a, g2b, deg0, deg1, bg2r,
                    Wp1, bp1r, Wp2, bp2r)
    return out
